# Initial kernel scaffold; baseline (speedup 1.0000x reference)
import jax
import jax.numpy as jnp
from jax.experimental import pallas as pl


def kernel(keys, values, lengths, new_keys, new_values, new_lengths):
    raise NotImplementedError("write your pallas kernel here")


# speedup vs baseline: 7.4231x; 7.4231x over previous
import functools

import jax
import jax.numpy as jnp
from jax.experimental import pallas as pl
from jax.experimental.pallas import tpu as pltpu

_BLK = 1024
_Q = 8


def _merge_copy_kernel(lengths_ref, new_lengths_ref,
                       k_ref, v_ref, nk_ref, nv_ref,
                       ok_ref, ov_ref, olen_ref):
    b = pl.program_id(0)
    j = pl.program_id(1)
    base = j * _BLK
    l = lengths_ref[b]
    nl = new_lengths_ref[b]

    ok_ref[...] = k_ref[...]
    ov_ref[...] = v_ref[...]

    for q in range(_Q):
        pos = l + q
        hit = (q < nl) & (pos >= base) & (pos < base + _BLK)

        @pl.when(hit)
        def _():
            off = pos - base
            ok_ref[0, pl.ds(off, 1), :, :] = nk_ref[0, pl.ds(q, 1), :, :]
            ov_ref[0, pl.ds(off, 1), :, :] = nv_ref[0, pl.ds(q, 1), :, :]

    @pl.when(j == 0)
    def _():
        olen_ref[b] = l + nl


@jax.jit
def kernel(keys, values, lengths, new_keys, new_values, new_lengths):
    B, L, H, D = keys.shape
    grid = (B, L // _BLK)

    kv_spec = pl.BlockSpec((1, _BLK, H, D), lambda b, j, *_: (b, j, 0, 0))
    new_spec = pl.BlockSpec((1, _Q, H, D), lambda b, j, *_: (b, 0, 0, 0))

    out_k, out_v, out_len = pl.pallas_call(
        _merge_copy_kernel,
        grid_spec=pltpu.PrefetchScalarGridSpec(
            num_scalar_prefetch=2,
            grid=grid,
            in_specs=[kv_spec, kv_spec, new_spec, new_spec],
            out_specs=[
                kv_spec,
                kv_spec,
                pl.BlockSpec(memory_space=pltpu.SMEM),
            ],
        ),
        out_shape=[
            jax.ShapeDtypeStruct((B, L, H, D), keys.dtype),
            jax.ShapeDtypeStruct((B, L, H, D), values.dtype),
            jax.ShapeDtypeStruct((B,), jnp.int32),
        ],
        compiler_params=pltpu.CompilerParams(
            dimension_semantics=("arbitrary", "arbitrary"),
        ),
    )(lengths, new_lengths, keys, values, new_keys, new_values)

    return (out_k, out_v, out_len)
